# baseline (speedup 1.0000x reference)
import jax
import jax.numpy as jnp
from jax import lax
from jax.experimental import pallas as pl
from jax.experimental.pallas import tpu as pltpu
from jax.experimental.pallas import tpu_sc as plsc

N = 10000
D = 128
E = 320000

NC = 2
NS = 16
NW = NC * NS
CH = 128
NCHUNK = 80
EPT = NCHUNK * CH
EP = EPT * NW
PAD = EP - E
HALF = NCHUNK // 2
N2 = 10240
RPT = N2 // NS


def _deg_body(col_hbm, ones_hbm, z128_hbm, acc_hbm,
              ones_v, cb0, cb1, cb2, cb3, is0, is1, is2, is3, acc_sh):
    c = lax.axis_index("c")
    s = lax.axis_index("s")
    wid = s * NC + c
    r0 = s * RPT
    cbs = (cb0, cb1, cb2, cb3)
    isems = (is0, is1, is2, is3)
    base = wid * NCHUNK

    pltpu.sync_copy(z128_hbm.at[pl.ds(r0, RPT)], acc_sh.at[pl.ds(r0, RPT)])
    pltpu.sync_copy(ones_hbm, ones_v)
    plsc.subcore_barrier()

    for q in range(4):
        pltpu.async_copy(col_hbm.at[base + q], cbs[q], isems[q])

    def grp(g, carry):
        for q in range(4):
            i = 4 * g + q
            pltpu.make_async_copy(col_hbm.at[base + i], cbs[q],
                                  isems[q]).wait()
            pltpu.sync_copy(ones_v, acc_sh.at[cbs[q]], add=True)

            @pl.when(i + 4 < NCHUNK)
            def _(i=i, q=q):
                pltpu.async_copy(col_hbm.at[base + i + 4], cbs[q], isems[q])
        return carry

    lax.fori_loop(0, NCHUNK // 4, grp, 0)
    plsc.subcore_barrier()
    pltpu.sync_copy(acc_sh.at[pl.ds(r0, RPT)], acc_hbm.at[c, pl.ds(r0, RPT)])


def _spmm_body(row_hbm, col_hbm, hp_hbm, z128_hbm, acc_hbm,
               ridx_all, buf0, buf1, cb0, cb1, cb2, cb3,
               gsem0, gsem1, is0, is1, is2, is3, acc_sh):
    c = lax.axis_index("c")
    s = lax.axis_index("s")
    wid = s * NC + c
    r0 = s * RPT
    bufs = (buf0, buf1)
    cbs = (cb0, cb1, cb2, cb3)
    gsems = (gsem0, gsem1)
    isems = (is0, is1, is2, is3)

    @pl.when(c == 0)
    def _():
        pltpu.sync_copy(hp_hbm.at[pl.ds(r0, RPT)], acc_sh.at[pl.ds(r0, RPT)])

    @pl.when(c != 0)
    def _():
        pltpu.sync_copy(z128_hbm.at[pl.ds(r0, RPT)], acc_sh.at[pl.ds(r0, RPT)])

    plsc.subcore_barrier()

    def gather(i, b):
        pltpu.async_copy(hp_hbm.at[ridx_all.at[i]], bufs[b], gsems[b])

    def gather_wait(i, b):
        pltpu.make_async_copy(hp_hbm.at[ridx_all.at[i]], bufs[b],
                              gsems[b]).wait()

    for h in range(2):
        base = wid * NCHUNK + h * HALF
        pltpu.sync_copy(row_hbm.at[pl.ds(base, HALF)], ridx_all)
        for q in range(4):
            pltpu.async_copy(col_hbm.at[base + q], cbs[q], isems[q])
        gather(0, 0)
        gather(1, 1)

        def grp(g, carry):
            for q in range(4):
                i = 4 * g + q
                b = q % 2
                gather_wait(i, b)
                pltpu.make_async_copy(col_hbm.at[base + i], cbs[q],
                                      isems[q]).wait()
                pltpu.sync_copy(bufs[b], acc_sh.at[cbs[q]], add=True)

                @pl.when(i + 2 < HALF)
                def _(i=i, b=b):
                    gather(i + 2, b)

                @pl.when(i + 4 < HALF)
                def _(i=i, q=q):
                    pltpu.async_copy(col_hbm.at[base + i + 4], cbs[q],
                                     isems[q])
            return carry

        lax.fori_loop(0, HALF // 4, grp, 0)
    plsc.subcore_barrier()
    pltpu.sync_copy(acc_sh.at[pl.ds(r0, RPT)], acc_hbm.at[c, pl.ds(r0, RPT)])


def _mm_body(x_ref, w_ref, d_ref, o_ref):
    h = jnp.dot(x_ref[...], w_ref[...], preferred_element_type=jnp.float32)
    deg = d_ref[0, :, 0:1] + d_ref[1, :, 0:1] + 1.0
    o_ref[...] = h * lax.rsqrt(deg)


def _comb_body(a0_ref, a1_ref, d_ref, b_ref, o_ref):
    deg = d_ref[0, :, 0:1] + d_ref[1, :, 0:1] + 1.0
    o_ref[...] = lax.rsqrt(deg) * (a0_ref[0] + a1_ref[0]) + b_ref[0]


@jax.jit
def kernel(x, edge_index, W, bias):
    row = edge_index[0]
    col = edge_index[1]
    row_p = jnp.concatenate(
        [row, jnp.zeros((PAD,), jnp.int32)]).reshape(NW * NCHUNK, CH)
    col_p = jnp.concatenate(
        [col, jnp.full((PAD,), N, jnp.int32)]).reshape(NW * NCHUNK, CH)
    x_p = jnp.concatenate([x, jnp.zeros((N2 - N, D), jnp.float32)])
    z128 = jnp.zeros((N2, D), jnp.float32)
    ones128 = jnp.ones((CH, D), jnp.float32)

    mesh = plsc.VectorSubcoreMesh(
        core_axis_name="c", subcore_axis_name="s",
        num_cores=NC, num_subcores=NS)

    degk = pl.kernel(
        _deg_body,
        out_type=jax.ShapeDtypeStruct((NC, N2, D), jnp.float32),
        mesh=mesh,
        scratch_types=[
            pltpu.VMEM((CH, D), jnp.float32),
            pltpu.VMEM((CH,), jnp.int32),
            pltpu.VMEM((CH,), jnp.int32),
            pltpu.VMEM((CH,), jnp.int32),
            pltpu.VMEM((CH,), jnp.int32),
            pltpu.SemaphoreType.DMA,
            pltpu.SemaphoreType.DMA,
            pltpu.SemaphoreType.DMA,
            pltpu.SemaphoreType.DMA,
            pltpu.VMEM_SHARED((N2, D), jnp.float32),
        ],
    )
    deg = degk(col_p, ones128, z128)

    mm_grid = 16
    bm = N2 // mm_grid
    hp = pl.pallas_call(
        _mm_body,
        grid=(mm_grid,),
        in_specs=[
            pl.BlockSpec((bm, D), lambda i: (i, 0)),
            pl.BlockSpec((D, D), lambda i: (0, 0)),
            pl.BlockSpec((2, bm, D), lambda i: (0, i, 0)),
        ],
        out_specs=pl.BlockSpec((bm, D), lambda i: (i, 0)),
        out_shape=jax.ShapeDtypeStruct((N2, D), jnp.float32),
    )(x_p, W, deg)

    spmm = pl.kernel(
        _spmm_body,
        out_type=jax.ShapeDtypeStruct((NC, N2, D), jnp.float32),
        mesh=mesh,
        scratch_types=[
            pltpu.VMEM((HALF, CH), jnp.int32),
            pltpu.VMEM((CH, D), jnp.float32),
            pltpu.VMEM((CH, D), jnp.float32),
            pltpu.VMEM((CH,), jnp.int32),
            pltpu.VMEM((CH,), jnp.int32),
            pltpu.VMEM((CH,), jnp.int32),
            pltpu.VMEM((CH,), jnp.int32),
            pltpu.SemaphoreType.DMA,
            pltpu.SemaphoreType.DMA,
            pltpu.SemaphoreType.DMA,
            pltpu.SemaphoreType.DMA,
            pltpu.SemaphoreType.DMA,
            pltpu.SemaphoreType.DMA,
            pltpu.VMEM_SHARED((N2, D), jnp.float32),
        ],
    )
    acc = spmm(row_p, col_p, hp, z128)

    out = pl.pallas_call(
        _comb_body,
        grid=(mm_grid,),
        in_specs=[
            pl.BlockSpec((1, bm, D), lambda i: (0, i, 0)),
            pl.BlockSpec((1, bm, D), lambda i: (1, i, 0)),
            pl.BlockSpec((2, bm, D), lambda i: (0, i, 0)),
            pl.BlockSpec((1, D), lambda i: (0, 0)),
        ],
        out_specs=pl.BlockSpec((bm, D), lambda i: (i, 0)),
        out_shape=jax.ShapeDtypeStruct((N, D), jnp.float32),
    )(acc, acc, deg, bias.reshape(1, D))
    return out
